# Initial kernel scaffold; baseline (speedup 1.0000x reference)
import jax
import jax.numpy as jnp
from jax.experimental import pallas as pl


def kernel(indices, text_prompt, token_prefix, token_suffix, tokenized_prompts, nc_token_prefix, nc_token_suffix, nc_tokenized_prompts):
    raise NotImplementedError("write your pallas kernel here")


# speedup vs baseline: 1.3312x; 1.3312x over previous
import functools

import jax
import jax.numpy as jnp
from jax import lax
from jax.experimental import pallas as pl
from jax.experimental.pallas import tpu as pltpu
from jax.experimental.pallas import tpu_sc as plsc

BATCH = 32
N_CLS = 100
CTX_DIM = 512
SEQ_LEN = 77
N_CTX = 12
TP = 3
POOL = 1000
SUF = SEQ_LEN - 1 - N_CTX * TP
NC_SUF = SEQ_LEN - 1 - N_CTX
CTX_ROWS = N_CTX * TP

CB = 25
NCB = N_CLS // CB
PB = 50

_GW = 12
_RPW = 8
_D = N_CTX * CTX_DIM


def _sc_gather(table, idx):
    mesh = plsc.VectorSubcoreMesh(core_axis_name="c", subcore_axis_name="s")

    @functools.partial(
        pl.kernel,
        mesh=mesh,
        out_type=jax.ShapeDtypeStruct((_GW * _RPW, _D), jnp.float32),
        scratch_types=[
            pltpu.VMEM((_RPW,), jnp.int32),
            pltpu.VMEM((_RPW, _D), jnp.float32),
            pltpu.SemaphoreType.DMA,
        ],
    )
    def k(table_hbm, idx_hbm, out_hbm, idx_v, rows_v, sem):
        wid = lax.axis_index("s") * 2 + lax.axis_index("c")

        @pl.when(wid < _GW)
        def _():
            base = wid * _RPW
            pltpu.sync_copy(idx_hbm.at[pl.ds(base, _RPW)], idx_v)
            pltpu.async_copy(table_hbm.at[idx_v], rows_v, sem).wait()
            pltpu.sync_copy(rows_v, out_hbm.at[pl.ds(base, _RPW)])

    return k(table, idx)


def _prompts_body(ctx_ref, pre_ref, suf_ref, tokp_ref, out_ref, tok_ref):
    out_ref[:, 0:1, :] = pre_ref[...]
    out_ref[:, 1:1 + CTX_ROWS, :] = jnp.broadcast_to(
        ctx_ref[...], (CB, CTX_ROWS, CTX_DIM))
    out_ref[:, 1 + CTX_ROWS:SEQ_LEN, :] = suf_ref[...]
    tok_ref[...] = tokp_ref[...]


def _assemble_prompts(ctx, token_prefix, token_suffix, tokp3):
    return pl.pallas_call(
        _prompts_body,
        grid=(NCB, BATCH),
        in_specs=[
            pl.BlockSpec((1, CTX_ROWS, CTX_DIM), lambda c, b: (b, 0, 0)),
            pl.BlockSpec((CB, 1, CTX_DIM), lambda c, b: (c, 0, 0)),
            pl.BlockSpec((CB, SUF, CTX_DIM), lambda c, b: (c, 0, 0)),
            pl.BlockSpec((CB, 1, SEQ_LEN), lambda c, b: (c, 0, 0)),
        ],
        out_specs=[
            pl.BlockSpec((CB, SEQ_LEN, CTX_DIM), lambda c, b: (b * NCB + c, 0, 0)),
            pl.BlockSpec((CB, 1, SEQ_LEN), lambda c, b: (b * NCB + c, 0, 0)),
        ],
        out_shape=[
            jax.ShapeDtypeStruct((BATCH * N_CLS, SEQ_LEN, CTX_DIM), jnp.float32),
            jax.ShapeDtypeStruct((BATCH * N_CLS, 1, SEQ_LEN), jnp.int32),
        ],
    )(ctx, token_prefix, token_suffix, tokp3)


def _nc_body(tp_ref, pre_ref, suf_ref, tok_ref, out_ref, otok_ref):
    out_ref[:, 0:1, :] = jnp.broadcast_to(pre_ref[...], (PB, 1, CTX_DIM))
    out_ref[:, 1:1 + N_CTX, :] = tp_ref[...]
    out_ref[:, 1 + N_CTX:SEQ_LEN, :] = jnp.broadcast_to(
        suf_ref[...], (PB, NC_SUF, CTX_DIM))
    otok_ref[...] = jnp.broadcast_to(tok_ref[...], (PB, 1, SEQ_LEN))


def _assemble_nc(text_prompt, nc_token_prefix, nc_token_suffix, nc_tok3):
    return pl.pallas_call(
        _nc_body,
        grid=(POOL // PB,),
        in_specs=[
            pl.BlockSpec((PB, N_CTX, CTX_DIM), lambda i: (i, 0, 0)),
            pl.BlockSpec((1, 1, CTX_DIM), lambda i: (0, 0, 0)),
            pl.BlockSpec((1, NC_SUF, CTX_DIM), lambda i: (0, 0, 0)),
            pl.BlockSpec((1, 1, SEQ_LEN), lambda i: (0, 0, 0)),
        ],
        out_specs=[
            pl.BlockSpec((PB, SEQ_LEN, CTX_DIM), lambda i: (i, 0, 0)),
            pl.BlockSpec((PB, 1, SEQ_LEN), lambda i: (i, 0, 0)),
        ],
        out_shape=[
            jax.ShapeDtypeStruct((POOL, SEQ_LEN, CTX_DIM), jnp.float32),
            jax.ShapeDtypeStruct((POOL, 1, SEQ_LEN), jnp.int32),
        ],
    )(text_prompt, nc_token_prefix, nc_token_suffix, nc_tok3)


def kernel(indices, text_prompt, token_prefix, token_suffix, tokenized_prompts,
           nc_token_prefix, nc_token_suffix, nc_tokenized_prompts):
    idx = indices.reshape(-1).astype(jnp.int32)
    table = text_prompt.reshape(POOL, _D)
    ctx = _sc_gather(table, idx).reshape(BATCH, CTX_ROWS, CTX_DIM)

    tokp3 = tokenized_prompts.reshape(N_CLS, 1, SEQ_LEN)
    prompts, tok3 = _assemble_prompts(ctx, token_prefix, token_suffix, tokp3)

    nc_tok3 = nc_tokenized_prompts.reshape(1, 1, SEQ_LEN)
    nc_prompts, nc_tok3o = _assemble_nc(
        text_prompt, nc_token_prefix, nc_token_suffix, nc_tok3)

    return (prompts,
            tok3.reshape(BATCH * N_CLS, SEQ_LEN),
            nc_prompts,
            nc_tok3o.reshape(POOL, SEQ_LEN))


# speedup vs baseline: 1.3621x; 1.0232x over previous
import functools

import jax
import jax.numpy as jnp
from jax import lax
from jax.experimental import pallas as pl
from jax.experimental.pallas import tpu as pltpu
from jax.experimental.pallas import tpu_sc as plsc

BATCH = 32
N_CLS = 100
CTX_DIM = 512
SEQ_LEN = 77
N_CTX = 12
TP = 3
POOL = 1000
SUF = SEQ_LEN - 1 - N_CTX * TP
NC_SUF = SEQ_LEN - 1 - N_CTX
CTX_ROWS = N_CTX * TP

CB = 50
NCB = N_CLS // CB
PB = 125

_GW = 12
_RPW = 8
_D = N_CTX * CTX_DIM


def _sc_gather(table, idx):
    mesh = plsc.VectorSubcoreMesh(core_axis_name="c", subcore_axis_name="s")

    @functools.partial(
        pl.kernel,
        mesh=mesh,
        out_type=jax.ShapeDtypeStruct((_GW * _RPW, _D), jnp.float32),
        scratch_types=[
            pltpu.VMEM((_RPW,), jnp.int32),
            pltpu.VMEM((_RPW, _D), jnp.float32),
            pltpu.SemaphoreType.DMA,
        ],
    )
    def k(table_hbm, idx_hbm, out_hbm, idx_v, rows_v, sem):
        wid = lax.axis_index("s") * 2 + lax.axis_index("c")

        @pl.when(wid < _GW)
        def _():
            base = wid * _RPW
            pltpu.sync_copy(idx_hbm.at[pl.ds(base, _RPW)], idx_v)
            pltpu.async_copy(table_hbm.at[idx_v], rows_v, sem).wait()
            pltpu.sync_copy(rows_v, out_hbm.at[pl.ds(base, _RPW)])

    return k(table, idx)


def _prompts_body(ctx_ref, pre_ref, suf_ref, tokp_ref, out_ref, tok_ref):
    out_ref[:, 0:1, :] = pre_ref[...]
    out_ref[:, 1:1 + CTX_ROWS, :] = jnp.broadcast_to(
        ctx_ref[...], (CB, CTX_ROWS, CTX_DIM))
    out_ref[:, 1 + CTX_ROWS:SEQ_LEN, :] = suf_ref[...]
    tok_ref[...] = tokp_ref[...]


def _assemble_prompts(ctx, token_prefix, token_suffix, tokp3):
    return pl.pallas_call(
        _prompts_body,
        grid=(NCB, BATCH),
        in_specs=[
            pl.BlockSpec((1, CTX_ROWS, CTX_DIM), lambda c, b: (b, 0, 0)),
            pl.BlockSpec((CB, 1, CTX_DIM), lambda c, b: (c, 0, 0)),
            pl.BlockSpec((CB, SUF, CTX_DIM), lambda c, b: (c, 0, 0)),
            pl.BlockSpec((CB, 1, SEQ_LEN), lambda c, b: (c, 0, 0)),
        ],
        out_specs=[
            pl.BlockSpec((CB, SEQ_LEN, CTX_DIM), lambda c, b: (b * NCB + c, 0, 0)),
            pl.BlockSpec((CB, 1, SEQ_LEN), lambda c, b: (b * NCB + c, 0, 0)),
        ],
        out_shape=[
            jax.ShapeDtypeStruct((BATCH * N_CLS, SEQ_LEN, CTX_DIM), jnp.float32),
            jax.ShapeDtypeStruct((BATCH * N_CLS, 1, SEQ_LEN), jnp.int32),
        ],
    )(ctx, token_prefix, token_suffix, tokp3)


def _nc_body(tp_ref, pre_ref, suf_ref, tok_ref, out_ref, otok_ref):
    out_ref[:, 0:1, :] = jnp.broadcast_to(pre_ref[...], (PB, 1, CTX_DIM))
    out_ref[:, 1:1 + N_CTX, :] = tp_ref[...]
    out_ref[:, 1 + N_CTX:SEQ_LEN, :] = jnp.broadcast_to(
        suf_ref[...], (PB, NC_SUF, CTX_DIM))
    otok_ref[...] = jnp.broadcast_to(tok_ref[...], (PB, 1, SEQ_LEN))


def _assemble_nc(text_prompt, nc_token_prefix, nc_token_suffix, nc_tok3):
    return pl.pallas_call(
        _nc_body,
        grid=(POOL // PB,),
        in_specs=[
            pl.BlockSpec((PB, N_CTX, CTX_DIM), lambda i: (i, 0, 0)),
            pl.BlockSpec((1, 1, CTX_DIM), lambda i: (0, 0, 0)),
            pl.BlockSpec((1, NC_SUF, CTX_DIM), lambda i: (0, 0, 0)),
            pl.BlockSpec((1, 1, SEQ_LEN), lambda i: (0, 0, 0)),
        ],
        out_specs=[
            pl.BlockSpec((PB, SEQ_LEN, CTX_DIM), lambda i: (i, 0, 0)),
            pl.BlockSpec((PB, 1, SEQ_LEN), lambda i: (i, 0, 0)),
        ],
        out_shape=[
            jax.ShapeDtypeStruct((POOL, SEQ_LEN, CTX_DIM), jnp.float32),
            jax.ShapeDtypeStruct((POOL, 1, SEQ_LEN), jnp.int32),
        ],
    )(text_prompt, nc_token_prefix, nc_token_suffix, nc_tok3)


def kernel(indices, text_prompt, token_prefix, token_suffix, tokenized_prompts,
           nc_token_prefix, nc_token_suffix, nc_tokenized_prompts):
    idx = indices.reshape(-1).astype(jnp.int32)
    table = text_prompt.reshape(POOL, _D)
    ctx = _sc_gather(table, idx).reshape(BATCH, CTX_ROWS, CTX_DIM)

    tokp3 = tokenized_prompts.reshape(N_CLS, 1, SEQ_LEN)
    prompts, tok3 = _assemble_prompts(ctx, token_prefix, token_suffix, tokp3)

    nc_tok3 = nc_tokenized_prompts.reshape(1, 1, SEQ_LEN)
    nc_prompts, nc_tok3o = _assemble_nc(
        text_prompt, nc_token_prefix, nc_token_suffix, nc_tok3)

    return (prompts,
            tok3.reshape(BATCH * N_CLS, SEQ_LEN),
            nc_prompts,
            nc_tok3o.reshape(POOL, SEQ_LEN))
